# baseline (speedup 1.0000x reference)
import functools

import jax
import jax.numpy as jnp
from jax import lax
from jax.experimental import pallas as pl
from jax.experimental.pallas import tpu as pltpu
from jax.experimental.pallas import tpu_sc as plsc

E = 100000
EP = 102400
D = 64
N = 204800
U = N // 2
NC = 2
NS = 16
L = 16
GU = 64
UNITS_PER_TILE = U // NS
NITER = UNITS_PER_TILE // GU
NCHUNK = 16
CH = EP // NCHUNK
W = CH // NS
TRASH = CH
ACC_ROWS = CH + 8
SUB = 16


def _scatter_body(grad_hbm, ide_hbm, ido_hbm, zero_hbm, out_hbm,
                  iebuf, iobuf, lbufe, lbufo, gbuf, bufa, bufb, bufc,
                  accl, accr):
    c = lax.axis_index("c")
    s = lax.axis_index("s")

    lo = s * W
    for k in range(NCHUNK // NC):
        base = c * (NCHUNK // NC * CH) + k * CH

        pltpu.sync_copy(zero_hbm.at[pl.ds(0, W)], accl.at[pl.ds(lo, W)])
        pltpu.sync_copy(zero_hbm.at[pl.ds(0, W)], accr.at[pl.ds(lo, W)])
        plsc.subcore_barrier()

        def compute_offsets(i, par):
            u0 = s * UNITS_PER_TILE + i * GU
            pltpu.sync_copy(ide_hbm.at[pl.ds(u0, GU)], iebuf)
            pltpu.sync_copy(ido_hbm.at[pl.ds(u0, GU)], iobuf)
            for j in range(GU // L):
                ve = iebuf[pl.ds(j * L, L)]
                oke = jnp.logical_and(
                    ve != 0,
                    jnp.logical_and(ve >= base, ve < base + CH))
                lbufe[par, pl.ds(j * L, L)] = jnp.where(
                    oke, ve - base, TRASH)
                vo = iobuf[pl.ds(j * L, L)]
                oko = jnp.logical_and(
                    vo != 0,
                    jnp.logical_and(vo >= base, vo < base + CH))
                lbufo[par, pl.ds(j * L, L)] = jnp.where(
                    oko, vo - base, TRASH)

        compute_offsets(0, 0)

        def step(i, carry):
            par = lax.rem(i, 2)
            u0 = s * UNITS_PER_TILE + i * GU
            pltpu.sync_copy(grad_hbm.at[pl.ds(u0, GU)], gbuf)
            compute_offsets(jnp.minimum(i + 1, NITER - 1), 1 - par)
            pltpu.sync_copy(gbuf, accl.at[lbufe.at[par]], add=True)
            pltpu.sync_copy(gbuf, accr.at[lbufo.at[par]], add=True)
            return carry

        lax.fori_loop(0, NITER, step, 0)
        plsc.subcore_barrier()

        def comb(t, carry):
            r0 = lo + t * SUB
            pltpu.sync_copy(accl.at[pl.ds(r0, SUB)], bufa)
            pltpu.sync_copy(accr.at[pl.ds(r0, SUB)], bufb)
            for r in range(SUB):
                for q in range(D // L):
                    bufc[r, pl.ds(q * L, L)] = (
                        bufa[r, pl.ds(q * L, L)]
                        + bufb[r, pl.ds(D + q * L, L)])
            pltpu.sync_copy(bufc, out_hbm.at[pl.ds(base + r0, SUB)])
            return carry

        lax.fori_loop(0, W // SUB, comb, 0)
        plsc.subcore_barrier()


_scatter = functools.partial(
    pl.kernel,
    mesh=plsc.VectorSubcoreMesh(core_axis_name="c", subcore_axis_name="s"),
    out_type=jax.ShapeDtypeStruct((EP, D), jnp.float32),
    scratch_types=[
        pltpu.VMEM((GU,), jnp.int32),
        pltpu.VMEM((GU,), jnp.int32),
        pltpu.VMEM((2, GU), jnp.int32),
        pltpu.VMEM((2, GU), jnp.int32),
        pltpu.VMEM((GU, 2 * D), jnp.float32),
        pltpu.VMEM((SUB, 2 * D), jnp.float32),
        pltpu.VMEM((SUB, 2 * D), jnp.float32),
        pltpu.VMEM((SUB, D), jnp.float32),
        pltpu.VMEM_SHARED((ACC_ROWS, 2 * D), jnp.float32),
        pltpu.VMEM_SHARED((ACC_ROWS, 2 * D), jnp.float32),
    ],
)(_scatter_body)


@jax.jit
def _run(grad2, idx_e, idx_o):
    zero = jnp.zeros((W, 2 * D), jnp.float32)
    padded = _scatter(grad2, idx_e, idx_o, zero)
    return padded[:E]


def kernel(grad_output, indices, num_embeddings):
    grad2 = grad_output.reshape(U, 2 * D)
    flat_idx = indices.reshape(N).astype(jnp.int32)
    return _run(grad2, flat_idx[0::2], flat_idx[1::2])


# speedup vs baseline: 1.2255x; 1.2255x over previous
import functools

import jax
import jax.numpy as jnp
from jax import lax
from jax.experimental import pallas as pl
from jax.experimental.pallas import tpu as pltpu
from jax.experimental.pallas import tpu_sc as plsc

E = 100000
EP = 102400
D = 64
N = 204800
U = N // 2
NC = 2
NS = 16
L = 16
GU = 64
UNITS_PER_TILE = U // NS
QU = 1600
NQ = UNITS_PER_TILE // QU
QITER = QU // GU
NCHUNK = 16
CH = EP // NCHUNK
W = CH // NS
TRASH = CH
ACC_ROWS = CH + 8
SUB = 8


def _scatter_body(grad_hbm, ide_hbm, ido_hbm, zero_hbm, out_hbm,
                  iebuf, iobuf, lbufe, lbufo, gbuf, bufa, bufb, bufc,
                  sem, accl, accr):
    c = lax.axis_index("c")
    s = lax.axis_index("s")

    lo = s * W
    u0t = s * UNITS_PER_TILE
    for k in range(NCHUNK // NC):
        base = c * (NCHUNK // NC * CH) + k * CH

        pltpu.sync_copy(zero_hbm.at[pl.ds(0, W)], accl.at[pl.ds(lo, W)])
        pltpu.sync_copy(zero_hbm.at[pl.ds(0, W)], accr.at[pl.ds(lo, W)])
        plsc.subcore_barrier()

        def compute_offsets(i, par):
            for j in range(GU // L):
                ve = iebuf[pl.ds(i * GU + j * L, L)]
                oke = jnp.logical_and(
                    ve != 0,
                    jnp.logical_and(ve >= base, ve < base + CH))
                lbufe[par, pl.ds(j * L, L)] = jnp.where(
                    oke, ve - base, TRASH)
                vo = iobuf[pl.ds(i * GU + j * L, L)]
                oko = jnp.logical_and(
                    vo != 0,
                    jnp.logical_and(vo >= base, vo < base + CH))
                lbufo[par, pl.ds(j * L, L)] = jnp.where(
                    oko, vo - base, TRASH)

        def quarter(q, carry):
            uq = u0t + q * QU
            pltpu.sync_copy(ide_hbm.at[pl.ds(uq, QU)], iebuf)
            pltpu.sync_copy(ido_hbm.at[pl.ds(uq, QU)], iobuf)
            compute_offsets(0, 0)

            def step(i, carry2):
                par = lax.rem(i, 2)
                pltpu.sync_copy(grad_hbm.at[pl.ds(uq + i * GU, GU)], gbuf)
                compute_offsets(jnp.minimum(i + 1, QITER - 1), 1 - par)
                hl = pltpu.async_copy(gbuf, accl.at[lbufe.at[par]], sem,
                                      add=True)
                hr = pltpu.async_copy(gbuf, accr.at[lbufo.at[par]], sem,
                                      add=True)
                hl.wait()
                hr.wait()
                return carry2

            lax.fori_loop(0, QITER, step, 0)
            return carry

        lax.fori_loop(0, NQ, quarter, 0)
        plsc.subcore_barrier()

        def comb(t, carry):
            r0 = lo + t * SUB
            pltpu.sync_copy(accl.at[pl.ds(r0, SUB)], bufa)
            pltpu.sync_copy(accr.at[pl.ds(r0, SUB)], bufb)
            for r in range(SUB):
                for qq in range(D // L):
                    bufc[r, pl.ds(qq * L, L)] = (
                        bufa[r, pl.ds(qq * L, L)]
                        + bufb[r, pl.ds(D + qq * L, L)])
            pltpu.sync_copy(bufc, out_hbm.at[pl.ds(base + r0, SUB)])
            return carry

        lax.fori_loop(0, W // SUB, comb, 0)
        plsc.subcore_barrier()


_scatter = functools.partial(
    pl.kernel,
    mesh=plsc.VectorSubcoreMesh(core_axis_name="c", subcore_axis_name="s"),
    out_type=jax.ShapeDtypeStruct((EP, D), jnp.float32),
    scratch_types=[
        pltpu.VMEM((QU,), jnp.int32),
        pltpu.VMEM((QU,), jnp.int32),
        pltpu.VMEM((2, GU), jnp.int32),
        pltpu.VMEM((2, GU), jnp.int32),
        pltpu.VMEM((GU, 2 * D), jnp.float32),
        pltpu.VMEM((SUB, 2 * D), jnp.float32),
        pltpu.VMEM((SUB, 2 * D), jnp.float32),
        pltpu.VMEM((SUB, D), jnp.float32),
        pltpu.SemaphoreType.DMA,
        pltpu.VMEM_SHARED((ACC_ROWS, 2 * D), jnp.float32),
        pltpu.VMEM_SHARED((ACC_ROWS, 2 * D), jnp.float32),
    ],
)(_scatter_body)


@jax.jit
def _run(grad2, idx_e, idx_o):
    zero = jnp.zeros((W, 2 * D), jnp.float32)
    padded = _scatter(grad2, idx_e, idx_o, zero)
    return padded[:E]


def kernel(grad_output, indices, num_embeddings):
    grad2 = grad_output.reshape(U, 2 * D)
    flat_idx = indices.reshape(N).astype(jnp.int32)
    return _run(grad2, flat_idx[0::2], flat_idx[1::2])


# speedup vs baseline: 1.2346x; 1.0074x over previous
import functools

import jax
import jax.numpy as jnp
from jax import lax
from jax.experimental import pallas as pl
from jax.experimental.pallas import tpu as pltpu
from jax.experimental.pallas import tpu_sc as plsc

E = 100000
EP = 102400
D = 64
N = 204800
U = N // 2
NC = 2
NS = 16
L = 16
GU = 32
UNITS_PER_TILE = U // NS
QU = 1600
NQ = UNITS_PER_TILE // QU
QITER = QU // GU
NCHUNK = 16
CH = EP // NCHUNK
W = CH // NS
TRASH = CH
ACC_ROWS = CH + 8
SUB = 8


def _scatter_body(grad_hbm, ide_hbm, ido_hbm, zero_hbm, out_hbm,
                  iebuf, iobuf, lbufe, lbufo, gbuf, bufa, bufb, bufc,
                  sem, gsem, accl, accr):
    c = lax.axis_index("c")
    s = lax.axis_index("s")

    lo = s * W
    u0t = s * UNITS_PER_TILE
    for k in range(NCHUNK // NC):
        base = c * (NCHUNK // NC * CH) + k * CH

        pltpu.sync_copy(zero_hbm.at[pl.ds(0, W)], accl.at[pl.ds(lo, W)])
        pltpu.sync_copy(zero_hbm.at[pl.ds(0, W)], accr.at[pl.ds(lo, W)])
        plsc.subcore_barrier()

        def compute_offsets(i, par):
            for j in range(GU // L):
                ve = iebuf[pl.ds(i * GU + j * L, L)]
                oke = jnp.logical_and(
                    ve != 0,
                    jnp.logical_and(ve >= base, ve < base + CH))
                lbufe[par, pl.ds(j * L, L)] = jnp.where(
                    oke, ve - base, TRASH)
                vo = iobuf[pl.ds(i * GU + j * L, L)]
                oko = jnp.logical_and(
                    vo != 0,
                    jnp.logical_and(vo >= base, vo < base + CH))
                lbufo[par, pl.ds(j * L, L)] = jnp.where(
                    oko, vo - base, TRASH)

        def quarter(q, carry):
            uq = u0t + q * QU
            pltpu.sync_copy(ide_hbm.at[pl.ds(uq, QU)], iebuf)
            pltpu.sync_copy(ido_hbm.at[pl.ds(uq, QU)], iobuf)
            compute_offsets(0, 0)
            pltpu.async_copy(grad_hbm.at[pl.ds(uq, GU)], gbuf.at[0], gsem)

            def step(i, carry2):
                par = lax.rem(i, 2)
                inext = jnp.minimum(i + 1, QITER - 1)
                pltpu.async_copy(grad_hbm.at[pl.ds(uq + inext * GU, GU)],
                                 gbuf.at[1 - par], gsem)
                compute_offsets(inext, 1 - par)
                pltpu.make_async_copy(grad_hbm.at[pl.ds(uq, GU)],
                                      gbuf.at[par], gsem).wait()
                hl = pltpu.async_copy(gbuf.at[par], accl.at[lbufe.at[par]],
                                      sem, add=True)
                hr = pltpu.async_copy(gbuf.at[par], accr.at[lbufo.at[par]],
                                      sem, add=True)
                hl.wait()
                hr.wait()
                return carry2

            lax.fori_loop(0, QITER, step, 0)
            pltpu.make_async_copy(grad_hbm.at[pl.ds(uq, GU)],
                                  gbuf.at[0], gsem).wait()
            return carry

        lax.fori_loop(0, NQ, quarter, 0)
        plsc.subcore_barrier()

        def comb(t, carry):
            r0 = lo + t * SUB
            pltpu.sync_copy(accl.at[pl.ds(r0, SUB)], bufa)
            pltpu.sync_copy(accr.at[pl.ds(r0, SUB)], bufb)
            for r in range(SUB):
                for qq in range(D // L):
                    bufc[r, pl.ds(qq * L, L)] = (
                        bufa[r, pl.ds(qq * L, L)]
                        + bufb[r, pl.ds(D + qq * L, L)])
            pltpu.sync_copy(bufc, out_hbm.at[pl.ds(base + r0, SUB)])
            return carry

        lax.fori_loop(0, W // SUB, comb, 0)
        plsc.subcore_barrier()


_scatter = functools.partial(
    pl.kernel,
    mesh=plsc.VectorSubcoreMesh(core_axis_name="c", subcore_axis_name="s"),
    out_type=jax.ShapeDtypeStruct((EP, D), jnp.float32),
    scratch_types=[
        pltpu.VMEM((QU,), jnp.int32),
        pltpu.VMEM((QU,), jnp.int32),
        pltpu.VMEM((2, GU), jnp.int32),
        pltpu.VMEM((2, GU), jnp.int32),
        pltpu.VMEM((2, GU, 2 * D), jnp.float32),
        pltpu.VMEM((SUB, 2 * D), jnp.float32),
        pltpu.VMEM((SUB, 2 * D), jnp.float32),
        pltpu.VMEM((SUB, D), jnp.float32),
        pltpu.SemaphoreType.DMA,
        pltpu.SemaphoreType.DMA,
        pltpu.VMEM_SHARED((ACC_ROWS, 2 * D), jnp.float32),
        pltpu.VMEM_SHARED((ACC_ROWS, 2 * D), jnp.float32),
    ],
)(_scatter_body)


@jax.jit
def _run(grad2, idx_e, idx_o):
    zero = jnp.zeros((W, 2 * D), jnp.float32)
    padded = _scatter(grad2, idx_e, idx_o, zero)
    return padded[:E]


def kernel(grad_output, indices, num_embeddings):
    grad2 = grad_output.reshape(U, 2 * D)
    flat_idx = indices.reshape(N).astype(jnp.int32)
    return _run(grad2, flat_idx[0::2], flat_idx[1::2])


# speedup vs baseline: 1.7979x; 1.4562x over previous
import functools

import jax
import jax.numpy as jnp
from jax import lax
from jax.experimental import pallas as pl
from jax.experimental.pallas import tpu as pltpu
from jax.experimental.pallas import tpu_sc as plsc

E = 100000
EP = 102400
D = 64
N = 204800
NC = 2
NS = 16
L = 16
GU = 32
UNITS_PER_TILE = N // NS
QU = 1600
NQ = UNITS_PER_TILE // QU
QITER = QU // GU
NCHUNK = 8
CH = EP // NCHUNK
W = CH // NS
TRASH = CH
ACC_ROWS = CH + 8
SUB = 8


def _scatter_body(grad_hbm, idx_hbm, zero_hbm, out_hbm,
                  ibuf, lbuf, gbuf, bufa, bufc, sem, gsem, acc):
    c = lax.axis_index("c")
    s = lax.axis_index("s")

    lo = s * W
    u0t = s * UNITS_PER_TILE
    for k in range(NCHUNK // NC):
        base = c * (NCHUNK // NC * CH) + k * CH

        pltpu.sync_copy(zero_hbm.at[pl.ds(0, W)], acc.at[pl.ds(lo, W)])
        plsc.subcore_barrier()

        def compute_offsets(i, par):
            for j in range(GU // L):
                v = ibuf[pl.ds(i * GU + j * L, L)]
                ok = jnp.logical_and(
                    v != 0,
                    jnp.logical_and(v >= base, v < base + CH))
                lbuf[par, pl.ds(j * L, L)] = jnp.where(
                    ok, v - base, TRASH)

        def batch(q, carry):
            uq = u0t + q * QU
            pltpu.sync_copy(idx_hbm.at[pl.ds(uq, QU)], ibuf)
            compute_offsets(0, 0)
            pltpu.async_copy(grad_hbm.at[pl.ds(uq, GU)], gbuf.at[0], gsem)

            def step(i, carry2):
                par = lax.rem(i, 2)
                inext = jnp.minimum(i + 1, QITER - 1)
                pltpu.async_copy(grad_hbm.at[pl.ds(uq + inext * GU, GU)],
                                 gbuf.at[1 - par], gsem)
                compute_offsets(inext, 1 - par)
                pltpu.make_async_copy(grad_hbm.at[pl.ds(uq, GU)],
                                      gbuf.at[par], gsem).wait()
                pltpu.async_copy(gbuf.at[par], acc.at[lbuf.at[par]],
                                 sem, add=True).wait()
                return carry2

            lax.fori_loop(0, QITER, step, 0)
            pltpu.make_async_copy(grad_hbm.at[pl.ds(uq, GU)],
                                  gbuf.at[0], gsem).wait()
            return carry

        lax.fori_loop(0, NQ, batch, 0)
        plsc.subcore_barrier()

        def comb(t, carry):
            r0 = lo + t * SUB
            pltpu.sync_copy(acc.at[pl.ds(r0, SUB)], bufa)
            for r in range(SUB):
                for qq in range(D // L):
                    bufc[r, pl.ds(qq * L, L)] = bufa[r, pl.ds(qq * L, L)]
            pltpu.sync_copy(bufc, out_hbm.at[pl.ds(base + r0, SUB)])
            return carry

        lax.fori_loop(0, W // SUB, comb, 0)
        plsc.subcore_barrier()


_scatter = functools.partial(
    pl.kernel,
    mesh=plsc.VectorSubcoreMesh(core_axis_name="c", subcore_axis_name="s"),
    out_type=jax.ShapeDtypeStruct((EP, D), jnp.float32),
    scratch_types=[
        pltpu.VMEM((QU,), jnp.int32),
        pltpu.VMEM((2, GU), jnp.int32),
        pltpu.VMEM((2, GU, 2 * D), jnp.float32),
        pltpu.VMEM((SUB, 2 * D), jnp.float32),
        pltpu.VMEM((SUB, D), jnp.float32),
        pltpu.SemaphoreType.DMA,
        pltpu.SemaphoreType.DMA,
        pltpu.VMEM_SHARED((ACC_ROWS, 2 * D), jnp.float32),
    ],
)(_scatter_body)


@jax.jit
def _run(grad2, flat_idx):
    zero = jnp.zeros((W, 2 * D), jnp.float32)
    padded = _scatter(grad2, flat_idx, zero)
    return padded[:E]


def kernel(grad_output, indices, num_embeddings):
    flat_grad = grad_output.reshape(N, D)
    grad2 = jnp.concatenate([flat_grad, flat_grad], axis=1)
    flat_idx = indices.reshape(N).astype(jnp.int32)
    return _run(grad2, flat_idx)
